# Initial kernel scaffold; baseline (speedup 1.0000x reference)
import jax
import jax.numpy as jnp
from jax.experimental import pallas as pl


def kernel(ctx_xyz, ctx_tokens, pred_xyz, pred_token, params):
    raise NotImplementedError("write your pallas kernel here")


# speedup vs baseline: 14.2638x; 14.2638x over previous
import functools

import jax
import jax.numpy as jnp
from jax import lax
from jax.experimental import pallas as pl
from jax.experimental.pallas import tpu as pltpu
from jax.experimental.pallas import tpu_sc as plsc

N_CTX = 2048
N_TGT = 8192
UP = 4
_BIG = 3.0e38
_DIAG = 1e10


def _mm(a, b):
    return jax.lax.dot_general(a, b, (((1,), (0,)), ((), ())),
                               preferred_element_type=jnp.float32)


def _mmT(a, b):
    return jax.lax.dot_general(a, b, (((1,), (1,)), ((), ())),
                               preferred_element_type=jnp.float32)


def _topk_cols(v, colids, k, n):
    cols = []
    for t in range(k):
        m = jnp.min(v, axis=1, keepdims=True)
        sel = jnp.min(jnp.where(v == m, colids, jnp.int32(n)), axis=1, keepdims=True)
        cols.append(sel)
        if t < k - 1:
            v = jnp.where(colids == sel, _BIG, v)
    return jnp.concatenate(cols, axis=1)


def _knn_body(x_ref, xf_ref, sqb_ref, sqr_ref, idx_ref, *, k, n, rb):
    i = pl.program_id(0)
    dotm = _mmT(x_ref[...], xf_ref[...])
    d2 = (sqb_ref[...] - 2.0 * dotm) + sqr_ref[...]
    colids = lax.broadcasted_iota(jnp.int32, (rb, n), 1)
    rowids = lax.broadcasted_iota(jnp.int32, (rb, n), 0) + i * rb
    d2 = jnp.where(colids == rowids, d2 + _DIAG, d2)
    idx_ref[...] = _topk_cols(d2, colids, k, n)


def _knn(x, sq, k, rb):
    n, f = x.shape
    return pl.pallas_call(
        functools.partial(_knn_body, k=k, n=n, rb=rb),
        grid=(n // rb,),
        in_specs=[
            pl.BlockSpec((rb, f), lambda i: (i, 0)),
            pl.BlockSpec((n, f), lambda i: (0, 0)),
            pl.BlockSpec((rb, 1), lambda i: (i, 0)),
            pl.BlockSpec((1, n), lambda i: (0, 0)),
        ],
        out_specs=pl.BlockSpec((rb, k), lambda i: (i, 0)),
        out_shape=jax.ShapeDtypeStruct((n, k), jnp.int32),
    )(x, x, sq[:, None], sq[None, :])


def _econv_body(x_ref, xg_ref, w1_ref, b1_ref, w2_ref, b2_ref,
                full_ref, selfk_ref, *, k, ks, f):
    xi = x_ref[...]
    w1 = w1_ref[...]
    b1 = b1_ref[...]
    w2 = w2_ref[...]
    b2 = b2_ref[...]

    def msg(xj):
        h = jnp.concatenate([xi, xj - xi], axis=1)
        return _mm(jax.nn.relu(_mm(h, w1) + b1), w2)

    msgs = [msg(xg_ref[:, j, :f]) for j in range(k)]
    mfull = msgs[0]
    for j in range(1, k):
        mfull = jnp.maximum(mfull, msgs[j])
    full_ref[...] = mfull + b2
    mself = msg(xi)
    for j in range(ks):
        mself = jnp.maximum(mself, msgs[j])
    selfk_ref[...] = mself + b2


def _econv(x, xg, w1, b1, w2, b2, ks, rb):
    n, f = x.shape
    k = xg.shape[1]
    h2 = w2.shape[1]
    return pl.pallas_call(
        functools.partial(_econv_body, k=k, ks=ks, f=f),
        grid=(n // rb,),
        in_specs=[
            pl.BlockSpec((rb, f), lambda i: (i, 0)),
            pl.BlockSpec((rb, k, xg.shape[2]), lambda i: (i, 0, 0)),
            pl.BlockSpec(w1.shape, lambda i: (0, 0)),
            pl.BlockSpec((1, w1.shape[1]), lambda i: (0, 0)),
            pl.BlockSpec(w2.shape, lambda i: (0, 0)),
            pl.BlockSpec((1, h2), lambda i: (0, 0)),
        ],
        out_specs=[
            pl.BlockSpec((rb, h2), lambda i: (i, 0)),
            pl.BlockSpec((rb, h2), lambda i: (i, 0)),
        ],
        out_shape=[
            jax.ShapeDtypeStruct((n, h2), jnp.float32),
            jax.ShapeDtypeStruct((n, h2), jnp.float32),
        ],
    )(x, xg, w1, b1[None, :], w2, b2[None, :])


def _ctxdef_body(feat_ref, xyz_ref, w1_ref, b1_ref, w2_ref, b2_ref, out_ref):
    h = jax.nn.relu(_mm(feat_ref[...], w1_ref[...]) + b1_ref[...])
    off = _mm(h, w2_ref[...]) + b2_ref[...]
    out_ref[...] = xyz_ref[...] + 0.05 * off


def _ctxdef(feat, xyz, w1, b1, w2, b2, rb):
    n, h = feat.shape
    return pl.pallas_call(
        _ctxdef_body,
        grid=(n // rb,),
        in_specs=[
            pl.BlockSpec((rb, h), lambda i: (i, 0)),
            pl.BlockSpec((rb, 3), lambda i: (i, 0)),
            pl.BlockSpec(w1.shape, lambda i: (0, 0)),
            pl.BlockSpec((1, w1.shape[1]), lambda i: (0, 0)),
            pl.BlockSpec(w2.shape, lambda i: (0, 0)),
            pl.BlockSpec((1, 3), lambda i: (0, 0)),
        ],
        out_specs=pl.BlockSpec((rb, 3), lambda i: (i, 0)),
        out_shape=jax.ShapeDtypeStruct((n, 3), jnp.float32),
    )(feat, xyz, w1, b1[None, :], w2, b2[None, :])


def _fold_body(h_ref, xyz_ref, w1_ref, b1_ref, w2_ref, b2_ref, w3_ref, b3_ref,
               out_ref):
    h1 = jax.nn.relu(_mm(h_ref[...], w1_ref[...]) + b1_ref[...])
    h2 = jax.nn.relu(_mm(h1, w2_ref[...]) + b2_ref[...])
    fold = _mm(h2, w3_ref[...]) + b3_ref[...]
    out_ref[...] = xyz_ref[...] + fold


def _fold(h, xyz0, w1, b1, w2, b2, w3, b3, rb):
    n, fin = h.shape
    return pl.pallas_call(
        _fold_body,
        grid=(n // rb,),
        in_specs=[
            pl.BlockSpec((rb, fin), lambda i: (i, 0)),
            pl.BlockSpec((rb, 3), lambda i: (i, 0)),
            pl.BlockSpec(w1.shape, lambda i: (0, 0)),
            pl.BlockSpec((1, w1.shape[1]), lambda i: (0, 0)),
            pl.BlockSpec(w2.shape, lambda i: (0, 0)),
            pl.BlockSpec((1, w2.shape[1]), lambda i: (0, 0)),
            pl.BlockSpec(w3.shape, lambda i: (0, 0)),
            pl.BlockSpec((1, 3), lambda i: (0, 0)),
        ],
        out_specs=pl.BlockSpec((rb, 3), lambda i: (i, 0)),
        out_shape=jax.ShapeDtypeStruct((n, 3), jnp.float32),
    )(h, xyz0, w1, b1[None, :], w2, b2[None, :], w3, b3[None, :])


def _refmsg_body(x_ref, xg_ref, xyz_ref, w1_ref, b1_ref, w2_ref, b2_ref,
                 out_ref, *, k, f):
    xi = x_ref[...]
    w1 = w1_ref[...]
    b1 = b1_ref[...]
    w2 = w2_ref[...]

    m = None
    for j in range(k):
        xj = xg_ref[:, j, :f]
        h = jnp.concatenate([xi, xj - xi], axis=1)
        mj = _mm(jax.nn.relu(_mm(h, w1) + b1), w2)
        m = mj if m is None else jnp.maximum(m, mj)
    out_ref[...] = xyz_ref[...] + (m + b2_ref[...])


def _refmsg(xcat, xg, xyz, w1, b1, w2, b2, rb):
    n, f = xcat.shape
    k = xg.shape[1]
    return pl.pallas_call(
        functools.partial(_refmsg_body, k=k, f=f),
        grid=(n // rb,),
        in_specs=[
            pl.BlockSpec((rb, f), lambda i: (i, 0)),
            pl.BlockSpec((rb, k, xg.shape[2]), lambda i: (i, 0, 0)),
            pl.BlockSpec((rb, 3), lambda i: (i, 0)),
            pl.BlockSpec(w1.shape, lambda i: (0, 0)),
            pl.BlockSpec((1, w1.shape[1]), lambda i: (0, 0)),
            pl.BlockSpec(w2.shape, lambda i: (0, 0)),
            pl.BlockSpec((1, 3), lambda i: (0, 0)),
        ],
        out_specs=pl.BlockSpec((rb, 3), lambda i: (i, 0)),
        out_shape=jax.ShapeDtypeStruct((n, 3), jnp.float32),
    )(xcat, xg, xyz, w1, b1[None, :], w2, b2[None, :])


def _sc_gather_pallas(table, idx):
    v, d = table.shape
    b = idx.shape[0]
    nw = 32
    bw = b // nw
    chunk = min(128, bw)
    nchunk = bw // chunk
    mesh = plsc.VectorSubcoreMesh(core_axis_name="c", subcore_axis_name="s")

    @functools.partial(
        pl.kernel,
        mesh=mesh,
        out_type=jax.ShapeDtypeStruct((b, d), jnp.float32),
        scratch_types=[
            pltpu.VMEM((chunk,), jnp.int32),
            pltpu.VMEM((chunk, d), jnp.float32),
            pltpu.SemaphoreType.DMA,
        ],
    )
    def gk(table_hbm, idx_hbm, out_hbm, idx_v, rows_v, sem):
        wid = lax.axis_index("s") * 2 + lax.axis_index("c")
        base = wid * bw
        for c in range(nchunk):
            off = base + c * chunk
            pltpu.sync_copy(idx_hbm.at[pl.ds(off, chunk)], idx_v)
            pltpu.async_copy(table_hbm.at[idx_v], rows_v, sem).wait()
            pltpu.sync_copy(rows_v, out_hbm.at[pl.ds(off, chunk)])

    return gk(table, idx)


def _sc_gather(table, idx):
    return _sc_gather_pallas(table, idx)


def _pad128(x):
    f = x.shape[1]
    pad = (-f) % 128
    return x if pad == 0 else jnp.pad(x, ((0, 0), (0, pad)))


def _sq(x):
    return jnp.sum(x * x, axis=1)


def kernel(ctx_xyz, ctx_tokens, pred_xyz, pred_token, params):
    B, P, C = ctx_tokens.shape
    x0 = ctx_tokens.reshape(-1, C)
    p = params

    idx1 = _knn(x0, _sq(x0), 16, rb=256)
    xg1 = _sc_gather(x0, idx1.reshape(-1)).reshape(N_CTX, 16, C)
    ctx_f1, tgt_f1 = _econv(x0, xg1,
                            p["conv1_l1"]["w"], p["conv1_l1"]["b"],
                            p["conv1_l2"]["w"], p["conv1_l2"]["b"],
                            ks=4, rb=256)

    idx2c = _knn(ctx_f1, _sq(ctx_f1), 8, rb=256)
    idx2t = _knn(tgt_f1, _sq(tgt_f1), 8, rb=256)
    xg2c = _sc_gather(_pad128(ctx_f1), idx2c.reshape(-1)).reshape(N_CTX, 8, 128)
    xg2t = _sc_gather(_pad128(tgt_f1), idx2t.reshape(-1)).reshape(N_CTX, 8, 128)
    ctx_feat, _ = _econv(ctx_f1, xg2c,
                         p["conv2_l1"]["w"], p["conv2_l1"]["b"],
                         p["conv2_l2"]["w"], p["conv2_l2"]["b"], ks=0, rb=256)
    _, tgt_feat_g = _econv(tgt_f1, xg2t,
                           p["conv2_l1"]["w"], p["conv2_l1"]["b"],
                           p["conv2_l2"]["w"], p["conv2_l2"]["b"], ks=2, rb=256)

    ctx_out = _ctxdef(ctx_feat, ctx_xyz.reshape(-1, 3),
                      p["ctxdef_l1"]["w"], p["ctxdef_l1"]["b"],
                      p["ctxdef_l2"]["w"], p["ctxdef_l2"]["b"], rb=256)

    noise = jax.random.normal(jax.random.key(42), (B, P * UP, 3), jnp.float32) * 0.02
    xyz0 = (jnp.repeat(pred_xyz, UP, axis=1) + noise).reshape(-1, 3)
    tgt_feat = jnp.repeat(tgt_feat_g, UP, axis=0)
    hfold = jnp.concatenate([xyz0, tgt_feat], axis=1)
    tgt_xyz1 = _fold(hfold, xyz0,
                     p["fold_l1"]["w"], p["fold_l1"]["b"],
                     p["fold_l2"]["w"], p["fold_l2"]["b"],
                     p["fold_l3"]["w"], p["fold_l3"]["b"], rb=512)

    idx3 = _knn(tgt_xyz1, _sq(tgt_xyz1), 16, rb=256)
    xcat = jnp.concatenate([tgt_feat, tgt_xyz1], axis=1)
    xg3 = _sc_gather(_pad128(xcat), idx3.reshape(-1)).reshape(N_TGT, 16, 128)
    tgt_out = _refmsg(xcat, xg3, tgt_xyz1,
                      p["ref_l1"]["w"], p["ref_l1"]["b"],
                      p["ref_l2"]["w"], p["ref_l2"]["b"], rb=256)

    return jnp.concatenate([ctx_out, tgt_out], axis=0)


# speedup vs baseline: 19.8097x; 1.3888x over previous
import functools

import jax
import jax.numpy as jnp
from jax import lax
from jax.experimental import pallas as pl
from jax.experimental.pallas import tpu as pltpu
from jax.experimental.pallas import tpu_sc as plsc

N_CTX = 2048
N_TGT = 8192
UP = 4
_BIG = 3.0e38
_DIAG = 1e10


def _mm(a, b):
    return jax.lax.dot_general(a, b, (((1,), (0,)), ((), ())),
                               preferred_element_type=jnp.float32)


def _mmT(a, b):
    return jax.lax.dot_general(a, b, (((1,), (1,)), ((), ())),
                               preferred_element_type=jnp.float32)


def _topk_cols(v, colids, k, n):
    cols = []
    for t in range(k):
        m = jnp.min(v, axis=1, keepdims=True)
        sel = jnp.min(jnp.where(v == m, colids, jnp.int32(n)), axis=1, keepdims=True)
        cols.append(sel)
        if t < k - 1:
            v = jnp.where(colids == sel, _BIG, v)
    return jnp.concatenate(cols, axis=1)


def _stride_min(v):
    while v.shape[1] > 128:
        h = v.shape[1] // 2
        v = jnp.minimum(v[:, :h], v[:, h:])
    return v


def _tile_lanes(m, n):
    return jnp.concatenate([m] * (n // 128), axis=1)


def _knn_body(x_ref, xf_ref, sqb_ref, sqr_ref, idx_ref, *, k, n, rb):
    i = pl.program_id(0)
    dotm = _mmT(x_ref[...], xf_ref[...])
    d2 = (sqb_ref[...] - 2.0 * dotm) + sqr_ref[...]
    colids = lax.broadcasted_iota(jnp.int32, (rb, n), 1)
    rowids = lax.broadcasted_iota(jnp.int32, (rb, n), 0) + i * rb
    d2 = jnp.where(colids == rowids, d2 + _DIAG, d2)

    ncand = 4
    cv, ci = [], []
    vw = d2
    for t in range(ncand):
        m = _stride_min(vw)
        w = jnp.where(vw == _tile_lanes(m, n), colids, jnp.int32(n))
        a = _stride_min(w)
        cv.append(m)
        ci.append(a)
        if t < ncand - 1:
            vw = jnp.where(colids == _tile_lanes(a, n), _BIG, vw)

    cols = []
    for t in range(k):
        m128 = jnp.minimum(jnp.minimum(cv[0], cv[1]), jnp.minimum(cv[2], cv[3]))
        m = jnp.min(m128, axis=1, keepdims=True)
        sel = None
        for r in range(ncand):
            w = jnp.min(jnp.where(cv[r] == m, ci[r], jnp.int32(n)),
                        axis=1, keepdims=True)
            sel = w if sel is None else jnp.minimum(sel, w)
        cols.append(sel)
        for r in range(ncand):
            cv[r] = jnp.where(ci[r] == sel, _BIG, cv[r])
    idx_ref[...] = jnp.concatenate(cols, axis=1)

    exh = (cv[0] == _BIG) & (cv[1] == _BIG) & (cv[2] == _BIG) & (cv[3] == _BIG)

    @pl.when(jnp.any(exh))
    def _fallback():
        idx_ref[...] = _topk_cols(d2, colids, k, n)


def _knn(x, sq, k, rb):
    n, f = x.shape
    return pl.pallas_call(
        functools.partial(_knn_body, k=k, n=n, rb=rb),
        grid=(n // rb,),
        in_specs=[
            pl.BlockSpec((rb, f), lambda i: (i, 0)),
            pl.BlockSpec((n, f), lambda i: (0, 0)),
            pl.BlockSpec((rb, 1), lambda i: (i, 0)),
            pl.BlockSpec((1, n), lambda i: (0, 0)),
        ],
        out_specs=pl.BlockSpec((rb, k), lambda i: (i, 0)),
        out_shape=jax.ShapeDtypeStruct((n, k), jnp.int32),
    )(x, x, sq[:, None], sq[None, :])


def _econv_body(x_ref, xg_ref, w1_ref, b1_ref, w2_ref, b2_ref,
                full_ref, selfk_ref, *, k, ks, f):
    xi = x_ref[...]
    w1 = w1_ref[...]
    b1 = b1_ref[...]
    w2 = w2_ref[...]
    b2 = b2_ref[...]

    def msg(xj):
        h = jnp.concatenate([xi, xj - xi], axis=1)
        return _mm(jax.nn.relu(_mm(h, w1) + b1), w2)

    msgs = [msg(xg_ref[:, j, :f]) for j in range(k)]
    mfull = msgs[0]
    for j in range(1, k):
        mfull = jnp.maximum(mfull, msgs[j])
    full_ref[...] = mfull + b2
    mself = msg(xi)
    for j in range(ks):
        mself = jnp.maximum(mself, msgs[j])
    selfk_ref[...] = mself + b2


def _econv(x, xg, w1, b1, w2, b2, ks, rb):
    n, f = x.shape
    k = xg.shape[1]
    h2 = w2.shape[1]
    return pl.pallas_call(
        functools.partial(_econv_body, k=k, ks=ks, f=f),
        grid=(n // rb,),
        in_specs=[
            pl.BlockSpec((rb, f), lambda i: (i, 0)),
            pl.BlockSpec((rb, k, xg.shape[2]), lambda i: (i, 0, 0)),
            pl.BlockSpec(w1.shape, lambda i: (0, 0)),
            pl.BlockSpec((1, w1.shape[1]), lambda i: (0, 0)),
            pl.BlockSpec(w2.shape, lambda i: (0, 0)),
            pl.BlockSpec((1, h2), lambda i: (0, 0)),
        ],
        out_specs=[
            pl.BlockSpec((rb, h2), lambda i: (i, 0)),
            pl.BlockSpec((rb, h2), lambda i: (i, 0)),
        ],
        out_shape=[
            jax.ShapeDtypeStruct((n, h2), jnp.float32),
            jax.ShapeDtypeStruct((n, h2), jnp.float32),
        ],
    )(x, xg, w1, b1[None, :], w2, b2[None, :])


def _ctxdef_body(feat_ref, xyz_ref, w1_ref, b1_ref, w2_ref, b2_ref, out_ref):
    h = jax.nn.relu(_mm(feat_ref[...], w1_ref[...]) + b1_ref[...])
    off = _mm(h, w2_ref[...]) + b2_ref[...]
    out_ref[...] = xyz_ref[...] + 0.05 * off


def _ctxdef(feat, xyz, w1, b1, w2, b2, rb):
    n, h = feat.shape
    return pl.pallas_call(
        _ctxdef_body,
        grid=(n // rb,),
        in_specs=[
            pl.BlockSpec((rb, h), lambda i: (i, 0)),
            pl.BlockSpec((rb, 3), lambda i: (i, 0)),
            pl.BlockSpec(w1.shape, lambda i: (0, 0)),
            pl.BlockSpec((1, w1.shape[1]), lambda i: (0, 0)),
            pl.BlockSpec(w2.shape, lambda i: (0, 0)),
            pl.BlockSpec((1, 3), lambda i: (0, 0)),
        ],
        out_specs=pl.BlockSpec((rb, 3), lambda i: (i, 0)),
        out_shape=jax.ShapeDtypeStruct((n, 3), jnp.float32),
    )(feat, xyz, w1, b1[None, :], w2, b2[None, :])


def _fold_body(h_ref, xyz_ref, w1_ref, b1_ref, w2_ref, b2_ref, w3_ref, b3_ref,
               out_ref):
    h1 = jax.nn.relu(_mm(h_ref[...], w1_ref[...]) + b1_ref[...])
    h2 = jax.nn.relu(_mm(h1, w2_ref[...]) + b2_ref[...])
    fold = _mm(h2, w3_ref[...]) + b3_ref[...]
    out_ref[...] = xyz_ref[...] + fold


def _fold(h, xyz0, w1, b1, w2, b2, w3, b3, rb):
    n, fin = h.shape
    return pl.pallas_call(
        _fold_body,
        grid=(n // rb,),
        in_specs=[
            pl.BlockSpec((rb, fin), lambda i: (i, 0)),
            pl.BlockSpec((rb, 3), lambda i: (i, 0)),
            pl.BlockSpec(w1.shape, lambda i: (0, 0)),
            pl.BlockSpec((1, w1.shape[1]), lambda i: (0, 0)),
            pl.BlockSpec(w2.shape, lambda i: (0, 0)),
            pl.BlockSpec((1, w2.shape[1]), lambda i: (0, 0)),
            pl.BlockSpec(w3.shape, lambda i: (0, 0)),
            pl.BlockSpec((1, 3), lambda i: (0, 0)),
        ],
        out_specs=pl.BlockSpec((rb, 3), lambda i: (i, 0)),
        out_shape=jax.ShapeDtypeStruct((n, 3), jnp.float32),
    )(h, xyz0, w1, b1[None, :], w2, b2[None, :], w3, b3[None, :])


def _refmsg_body(x_ref, xg_ref, xyz_ref, w1_ref, b1_ref, w2_ref, b2_ref,
                 out_ref, *, k, f):
    xi = x_ref[...]
    w1 = w1_ref[...]
    b1 = b1_ref[...]
    w2 = w2_ref[...]

    m = None
    for j in range(k):
        xj = xg_ref[:, j, :f]
        h = jnp.concatenate([xi, xj - xi], axis=1)
        mj = _mm(jax.nn.relu(_mm(h, w1) + b1), w2)
        m = mj if m is None else jnp.maximum(m, mj)
    out_ref[...] = xyz_ref[...] + (m + b2_ref[...])


def _refmsg(xcat, xg, xyz, w1, b1, w2, b2, rb):
    n, f = xcat.shape
    k = xg.shape[1]
    return pl.pallas_call(
        functools.partial(_refmsg_body, k=k, f=f),
        grid=(n // rb,),
        in_specs=[
            pl.BlockSpec((rb, f), lambda i: (i, 0)),
            pl.BlockSpec((rb, k, xg.shape[2]), lambda i: (i, 0, 0)),
            pl.BlockSpec((rb, 3), lambda i: (i, 0)),
            pl.BlockSpec(w1.shape, lambda i: (0, 0)),
            pl.BlockSpec((1, w1.shape[1]), lambda i: (0, 0)),
            pl.BlockSpec(w2.shape, lambda i: (0, 0)),
            pl.BlockSpec((1, 3), lambda i: (0, 0)),
        ],
        out_specs=pl.BlockSpec((rb, 3), lambda i: (i, 0)),
        out_shape=jax.ShapeDtypeStruct((n, 3), jnp.float32),
    )(xcat, xg, xyz, w1, b1[None, :], w2, b2[None, :])


def _sc_gather_pallas(table, idx):
    v, d = table.shape
    b = idx.shape[0]
    nw = 32
    bw = b // nw
    chunk = min(128, bw)
    nchunk = bw // chunk
    mesh = plsc.VectorSubcoreMesh(core_axis_name="c", subcore_axis_name="s")

    @functools.partial(
        pl.kernel,
        mesh=mesh,
        out_type=jax.ShapeDtypeStruct((b, d), jnp.float32),
        scratch_types=[
            pltpu.VMEM((chunk,), jnp.int32),
            pltpu.VMEM((chunk, d), jnp.float32),
            pltpu.SemaphoreType.DMA,
        ],
    )
    def gk(table_hbm, idx_hbm, out_hbm, idx_v, rows_v, sem):
        wid = lax.axis_index("s") * 2 + lax.axis_index("c")
        base = wid * bw
        for c in range(nchunk):
            off = base + c * chunk
            pltpu.sync_copy(idx_hbm.at[pl.ds(off, chunk)], idx_v)
            pltpu.async_copy(table_hbm.at[idx_v], rows_v, sem).wait()
            pltpu.sync_copy(rows_v, out_hbm.at[pl.ds(off, chunk)])

    return gk(table, idx)


def _sc_gather(table, idx):
    return _sc_gather_pallas(table, idx)


def _pad128(x):
    f = x.shape[1]
    pad = (-f) % 128
    return x if pad == 0 else jnp.pad(x, ((0, 0), (0, pad)))


def _sq(x):
    return jnp.sum(x * x, axis=1)


def kernel(ctx_xyz, ctx_tokens, pred_xyz, pred_token, params):
    B, P, C = ctx_tokens.shape
    x0 = ctx_tokens.reshape(-1, C)
    p = params

    idx1 = _knn(x0, _sq(x0), 16, rb=256)
    xg1 = _sc_gather(x0, idx1.reshape(-1)).reshape(N_CTX, 16, C)
    ctx_f1, tgt_f1 = _econv(x0, xg1,
                            p["conv1_l1"]["w"], p["conv1_l1"]["b"],
                            p["conv1_l2"]["w"], p["conv1_l2"]["b"],
                            ks=4, rb=256)

    idx2c = _knn(ctx_f1, _sq(ctx_f1), 8, rb=256)
    idx2t = _knn(tgt_f1, _sq(tgt_f1), 8, rb=256)
    xg2c = _sc_gather(_pad128(ctx_f1), idx2c.reshape(-1)).reshape(N_CTX, 8, 128)
    xg2t = _sc_gather(_pad128(tgt_f1), idx2t.reshape(-1)).reshape(N_CTX, 8, 128)
    ctx_feat, _ = _econv(ctx_f1, xg2c,
                         p["conv2_l1"]["w"], p["conv2_l1"]["b"],
                         p["conv2_l2"]["w"], p["conv2_l2"]["b"], ks=0, rb=256)
    _, tgt_feat_g = _econv(tgt_f1, xg2t,
                           p["conv2_l1"]["w"], p["conv2_l1"]["b"],
                           p["conv2_l2"]["w"], p["conv2_l2"]["b"], ks=2, rb=256)

    ctx_out = _ctxdef(ctx_feat, ctx_xyz.reshape(-1, 3),
                      p["ctxdef_l1"]["w"], p["ctxdef_l1"]["b"],
                      p["ctxdef_l2"]["w"], p["ctxdef_l2"]["b"], rb=256)

    noise = jax.random.normal(jax.random.key(42), (B, P * UP, 3), jnp.float32) * 0.02
    xyz0 = (jnp.repeat(pred_xyz, UP, axis=1) + noise).reshape(-1, 3)
    tgt_feat = jnp.repeat(tgt_feat_g, UP, axis=0)
    hfold = jnp.concatenate([xyz0, tgt_feat], axis=1)
    tgt_xyz1 = _fold(hfold, xyz0,
                     p["fold_l1"]["w"], p["fold_l1"]["b"],
                     p["fold_l2"]["w"], p["fold_l2"]["b"],
                     p["fold_l3"]["w"], p["fold_l3"]["b"], rb=512)

    idx3 = _knn(tgt_xyz1, _sq(tgt_xyz1), 16, rb=256)
    xcat = jnp.concatenate([tgt_feat, tgt_xyz1], axis=1)
    xg3 = _sc_gather(_pad128(xcat), idx3.reshape(-1)).reshape(N_TGT, 16, 128)
    tgt_out = _refmsg(xcat, xg3, tgt_xyz1,
                      p["ref_l1"]["w"], p["ref_l1"]["b"],
                      p["ref_l2"]["w"], p["ref_l2"]["b"], rb=256)

    return jnp.concatenate([ctx_out, tgt_out], axis=0)


# speedup vs baseline: 20.2665x; 1.0231x over previous
import functools

import jax
import jax.numpy as jnp
from jax import lax
from jax.experimental import pallas as pl
from jax.experimental.pallas import tpu as pltpu
from jax.experimental.pallas import tpu_sc as plsc

N_CTX = 2048
N_TGT = 8192
UP = 4
_BIG = 3.0e38
_DIAG = 1e10


def _mm(a, b):
    return jax.lax.dot_general(a, b, (((1,), (0,)), ((), ())),
                               preferred_element_type=jnp.float32)


def _mmT(a, b):
    return jax.lax.dot_general(a, b, (((1,), (1,)), ((), ())),
                               preferred_element_type=jnp.float32)


def _topk_cols(v, colids, k, n):
    cols = []
    for t in range(k):
        m = jnp.min(v, axis=1, keepdims=True)
        sel = jnp.min(jnp.where(v == m, colids, jnp.int32(n)), axis=1, keepdims=True)
        cols.append(sel)
        if t < k - 1:
            v = jnp.where(colids == sel, _BIG, v)
    return jnp.concatenate(cols, axis=1)


def _stride_min(v):
    while v.shape[1] > 128:
        h = v.shape[1] // 2
        v = jnp.minimum(v[:, :h], v[:, h:])
    return v


def _tile_lanes(m, n):
    return jnp.concatenate([m] * (n // 128), axis=1)


def _knn_body(x_ref, xf_ref, sqb_ref, sqr_ref, idx_ref, *, k, n, rb):
    i = pl.program_id(0)
    dotm = _mmT(x_ref[...], xf_ref[...])
    d2 = (sqb_ref[...] - 2.0 * dotm) + sqr_ref[...]
    colids = lax.broadcasted_iota(jnp.int32, (rb, n), 1)
    rowids = lax.broadcasted_iota(jnp.int32, (rb, n), 0) + i * rb
    d2 = jnp.where(colids == rowids, d2 + _DIAG, d2)

    ncand = 4
    cv, ci = [], []
    vw = d2
    for t in range(ncand):
        m = _stride_min(vw)
        w = jnp.where(vw == _tile_lanes(m, n), colids, jnp.int32(n))
        a = _stride_min(w)
        cv.append(m)
        ci.append(a)
        if t < ncand - 1:
            vw = jnp.where(colids == _tile_lanes(a, n), _BIG, vw)

    cols = []
    for t in range(k):
        m128 = jnp.minimum(jnp.minimum(cv[0], cv[1]), jnp.minimum(cv[2], cv[3]))
        m = jnp.min(m128, axis=1, keepdims=True)
        sel = None
        for r in range(ncand):
            w = jnp.min(jnp.where(cv[r] == m, ci[r], jnp.int32(n)),
                        axis=1, keepdims=True)
            sel = w if sel is None else jnp.minimum(sel, w)
        cols.append(sel)
        for r in range(ncand):
            cv[r] = jnp.where(ci[r] == sel, _BIG, cv[r])
    idx_ref[...] = jnp.concatenate(cols, axis=1)

    exh = (cv[0] == _BIG) & (cv[1] == _BIG) & (cv[2] == _BIG) & (cv[3] == _BIG)

    @pl.when(jnp.any(exh))
    def _fallback():
        idx_ref[...] = _topk_cols(d2, colids, k, n)


def _knn(x, sq, k, rb):
    n, f = x.shape
    return pl.pallas_call(
        functools.partial(_knn_body, k=k, n=n, rb=rb),
        grid=(n // rb,),
        in_specs=[
            pl.BlockSpec((rb, f), lambda i: (i, 0)),
            pl.BlockSpec((n, f), lambda i: (0, 0)),
            pl.BlockSpec((rb, 1), lambda i: (i, 0)),
            pl.BlockSpec((1, n), lambda i: (0, 0)),
        ],
        out_specs=pl.BlockSpec((rb, k), lambda i: (i, 0)),
        out_shape=jax.ShapeDtypeStruct((n, k), jnp.int32),
    )(x, x, sq[:, None], sq[None, :])


def _econv_body(x_ref, xg_ref, w1_ref, b1_ref, w2_ref, b2_ref,
                full_ref, selfk_ref, *, k, ks, f):
    xi = x_ref[...]
    w1 = w1_ref[...]
    b1 = b1_ref[...]
    w2 = w2_ref[...]
    b2 = b2_ref[...]

    def msg(xj):
        h = jnp.concatenate([xi, xj - xi], axis=1)
        return _mm(jax.nn.relu(_mm(h, w1) + b1), w2)

    msgs = [msg(xg_ref[:, j, :f]) for j in range(k)]
    mfull = msgs[0]
    for j in range(1, k):
        mfull = jnp.maximum(mfull, msgs[j])
    full_ref[...] = mfull + b2
    mself = msg(xi)
    for j in range(ks):
        mself = jnp.maximum(mself, msgs[j])
    selfk_ref[...] = mself + b2


def _econv(x, xg, w1, b1, w2, b2, ks, rb):
    n, f = x.shape
    k = xg.shape[1]
    h2 = w2.shape[1]
    return pl.pallas_call(
        functools.partial(_econv_body, k=k, ks=ks, f=f),
        grid=(n // rb,),
        in_specs=[
            pl.BlockSpec((rb, f), lambda i: (i, 0)),
            pl.BlockSpec((rb, k, xg.shape[2]), lambda i: (i, 0, 0)),
            pl.BlockSpec(w1.shape, lambda i: (0, 0)),
            pl.BlockSpec((1, w1.shape[1]), lambda i: (0, 0)),
            pl.BlockSpec(w2.shape, lambda i: (0, 0)),
            pl.BlockSpec((1, h2), lambda i: (0, 0)),
        ],
        out_specs=[
            pl.BlockSpec((rb, h2), lambda i: (i, 0)),
            pl.BlockSpec((rb, h2), lambda i: (i, 0)),
        ],
        out_shape=[
            jax.ShapeDtypeStruct((n, h2), jnp.float32),
            jax.ShapeDtypeStruct((n, h2), jnp.float32),
        ],
    )(x, xg, w1, b1[None, :], w2, b2[None, :])


def _ctxdef_body(feat_ref, xyz_ref, w1_ref, b1_ref, w2_ref, b2_ref, out_ref):
    h = jax.nn.relu(_mm(feat_ref[...], w1_ref[...]) + b1_ref[...])
    off = _mm(h, w2_ref[...]) + b2_ref[...]
    out_ref[...] = xyz_ref[...] + 0.05 * off


def _ctxdef(feat, xyz, w1, b1, w2, b2, rb):
    n, h = feat.shape
    return pl.pallas_call(
        _ctxdef_body,
        grid=(n // rb,),
        in_specs=[
            pl.BlockSpec((rb, h), lambda i: (i, 0)),
            pl.BlockSpec((rb, 3), lambda i: (i, 0)),
            pl.BlockSpec(w1.shape, lambda i: (0, 0)),
            pl.BlockSpec((1, w1.shape[1]), lambda i: (0, 0)),
            pl.BlockSpec(w2.shape, lambda i: (0, 0)),
            pl.BlockSpec((1, 3), lambda i: (0, 0)),
        ],
        out_specs=pl.BlockSpec((rb, 3), lambda i: (i, 0)),
        out_shape=jax.ShapeDtypeStruct((n, 3), jnp.float32),
    )(feat, xyz, w1, b1[None, :], w2, b2[None, :])


def _fold_body(h_ref, xyz_ref, w1_ref, b1_ref, w2_ref, b2_ref, w3_ref, b3_ref,
               out_ref):
    h1 = jax.nn.relu(_mm(h_ref[...], w1_ref[...]) + b1_ref[...])
    h2 = jax.nn.relu(_mm(h1, w2_ref[...]) + b2_ref[...])
    fold = _mm(h2, w3_ref[...]) + b3_ref[...]
    out_ref[...] = xyz_ref[...] + fold


def _fold(h, xyz0, w1, b1, w2, b2, w3, b3, rb):
    n, fin = h.shape
    return pl.pallas_call(
        _fold_body,
        grid=(n // rb,),
        in_specs=[
            pl.BlockSpec((rb, fin), lambda i: (i, 0)),
            pl.BlockSpec((rb, 3), lambda i: (i, 0)),
            pl.BlockSpec(w1.shape, lambda i: (0, 0)),
            pl.BlockSpec((1, w1.shape[1]), lambda i: (0, 0)),
            pl.BlockSpec(w2.shape, lambda i: (0, 0)),
            pl.BlockSpec((1, w2.shape[1]), lambda i: (0, 0)),
            pl.BlockSpec(w3.shape, lambda i: (0, 0)),
            pl.BlockSpec((1, 3), lambda i: (0, 0)),
        ],
        out_specs=pl.BlockSpec((rb, 3), lambda i: (i, 0)),
        out_shape=jax.ShapeDtypeStruct((n, 3), jnp.float32),
    )(h, xyz0, w1, b1[None, :], w2, b2[None, :], w3, b3[None, :])


def _refmsg_body(x_ref, xg_ref, xyz_ref, w1_ref, b1_ref, w2_ref, b2_ref,
                 out_ref, *, k, f):
    xi = x_ref[...]
    w1 = w1_ref[...]
    b1 = b1_ref[...]
    w2 = w2_ref[...]

    m = None
    for j in range(k):
        xj = xg_ref[:, j, :f]
        h = jnp.concatenate([xi, xj - xi], axis=1)
        mj = _mm(jax.nn.relu(_mm(h, w1) + b1), w2)
        m = mj if m is None else jnp.maximum(m, mj)
    out_ref[...] = xyz_ref[...] + (m + b2_ref[...])


def _refmsg(xcat, xg, xyz, w1, b1, w2, b2, rb):
    n, f = xcat.shape
    k = xg.shape[1]
    return pl.pallas_call(
        functools.partial(_refmsg_body, k=k, f=f),
        grid=(n // rb,),
        in_specs=[
            pl.BlockSpec((rb, f), lambda i: (i, 0)),
            pl.BlockSpec((rb, k, xg.shape[2]), lambda i: (i, 0, 0)),
            pl.BlockSpec((rb, 3), lambda i: (i, 0)),
            pl.BlockSpec(w1.shape, lambda i: (0, 0)),
            pl.BlockSpec((1, w1.shape[1]), lambda i: (0, 0)),
            pl.BlockSpec(w2.shape, lambda i: (0, 0)),
            pl.BlockSpec((1, 3), lambda i: (0, 0)),
        ],
        out_specs=pl.BlockSpec((rb, 3), lambda i: (i, 0)),
        out_shape=jax.ShapeDtypeStruct((n, 3), jnp.float32),
    )(xcat, xg, xyz, w1, b1[None, :], w2, b2[None, :])


def _sc_gather_pallas(table, idx):
    v, d = table.shape
    b = idx.shape[0]
    nw = 32
    bw = b // nw
    chunk = min(128, bw)
    nchunk = bw // chunk
    mesh = plsc.VectorSubcoreMesh(core_axis_name="c", subcore_axis_name="s")

    @functools.partial(
        pl.kernel,
        mesh=mesh,
        out_type=jax.ShapeDtypeStruct((b, d), jnp.float32),
        scratch_types=[
            pltpu.VMEM((2, chunk), jnp.int32),
            pltpu.VMEM((2, chunk, d), jnp.float32),
            pltpu.SemaphoreType.DMA,
            pltpu.SemaphoreType.DMA,
            pltpu.SemaphoreType.DMA,
            pltpu.SemaphoreType.DMA,
            pltpu.SemaphoreType.DMA,
        ],
    )
    def gk(table_hbm, idx_hbm, out_hbm, idx_v, rows_v, si, sg0, sg1, ss0, ss1):
        wid = lax.axis_index("s") * 2 + lax.axis_index("c")
        base = wid * bw
        sg = (sg0, sg1)
        ss = (ss0, ss1)
        pltpu.sync_copy(idx_hbm.at[pl.ds(base, chunk)], idx_v.at[0])
        g = [pltpu.async_copy(table_hbm.at[idx_v.at[0]], rows_v.at[0], sg[0]), None]
        s = [None, None]
        for c in range(nchunk):
            cur = c & 1
            nxt = cur ^ 1
            if c + 1 < nchunk:
                ic = pltpu.async_copy(
                    idx_hbm.at[pl.ds(base + (c + 1) * chunk, chunk)],
                    idx_v.at[nxt], si)
                if s[nxt] is not None:
                    s[nxt].wait()
                    s[nxt] = None
                ic.wait()
                g[nxt] = pltpu.async_copy(table_hbm.at[idx_v.at[nxt]],
                                          rows_v.at[nxt], sg[nxt])
            g[cur].wait()
            s[cur] = pltpu.async_copy(rows_v.at[cur],
                                      out_hbm.at[pl.ds(base + c * chunk, chunk)],
                                      ss[cur])
        for bb in range(2):
            if s[bb] is not None:
                s[bb].wait()

    return gk(table, idx)


def _sc_gather(table, idx):
    return _sc_gather_pallas(table, idx)


def _pad128(x):
    f = x.shape[1]
    pad = (-f) % 128
    return x if pad == 0 else jnp.pad(x, ((0, 0), (0, pad)))


def _sq(x):
    return jnp.sum(x * x, axis=1)


def kernel(ctx_xyz, ctx_tokens, pred_xyz, pred_token, params):
    B, P, C = ctx_tokens.shape
    x0 = ctx_tokens.reshape(-1, C)
    p = params

    idx1 = _knn(x0, _sq(x0), 16, rb=256)
    xg1 = _sc_gather(x0, idx1.reshape(-1)).reshape(N_CTX, 16, C)
    ctx_f1, tgt_f1 = _econv(x0, xg1,
                            p["conv1_l1"]["w"], p["conv1_l1"]["b"],
                            p["conv1_l2"]["w"], p["conv1_l2"]["b"],
                            ks=4, rb=256)

    idx2c = _knn(ctx_f1, _sq(ctx_f1), 8, rb=256)
    idx2t = _knn(tgt_f1, _sq(tgt_f1), 8, rb=256)
    xg2c = _sc_gather(_pad128(ctx_f1), idx2c.reshape(-1)).reshape(N_CTX, 8, 128)
    xg2t = _sc_gather(_pad128(tgt_f1), idx2t.reshape(-1)).reshape(N_CTX, 8, 128)
    ctx_feat, _ = _econv(ctx_f1, xg2c,
                         p["conv2_l1"]["w"], p["conv2_l1"]["b"],
                         p["conv2_l2"]["w"], p["conv2_l2"]["b"], ks=0, rb=256)
    _, tgt_feat_g = _econv(tgt_f1, xg2t,
                           p["conv2_l1"]["w"], p["conv2_l1"]["b"],
                           p["conv2_l2"]["w"], p["conv2_l2"]["b"], ks=2, rb=256)

    ctx_out = _ctxdef(ctx_feat, ctx_xyz.reshape(-1, 3),
                      p["ctxdef_l1"]["w"], p["ctxdef_l1"]["b"],
                      p["ctxdef_l2"]["w"], p["ctxdef_l2"]["b"], rb=256)

    noise = jax.random.normal(jax.random.key(42), (B, P * UP, 3), jnp.float32) * 0.02
    xyz0 = (jnp.repeat(pred_xyz, UP, axis=1) + noise).reshape(-1, 3)
    tgt_feat = jnp.repeat(tgt_feat_g, UP, axis=0)
    hfold = jnp.concatenate([xyz0, tgt_feat], axis=1)
    tgt_xyz1 = _fold(hfold, xyz0,
                     p["fold_l1"]["w"], p["fold_l1"]["b"],
                     p["fold_l2"]["w"], p["fold_l2"]["b"],
                     p["fold_l3"]["w"], p["fold_l3"]["b"], rb=512)

    idx3 = _knn(tgt_xyz1, _sq(tgt_xyz1), 16, rb=256)
    xcat = jnp.concatenate([tgt_feat, tgt_xyz1], axis=1)
    xg3 = _sc_gather(_pad128(xcat), idx3.reshape(-1)).reshape(N_TGT, 16, 128)
    tgt_out = _refmsg(xcat, xg3, tgt_xyz1,
                      p["ref_l1"]["w"], p["ref_l1"]["b"],
                      p["ref_l2"]["w"], p["ref_l2"]["b"], rb=256)

    return jnp.concatenate([ctx_out, tgt_out], axis=0)
